# baseline (speedup 1.0000x reference)
import functools
import math as _math

import jax
import jax.numpy as jnp
from jax import lax
from jax.experimental import pallas as pl
from jax.experimental.pallas import tpu as pltpu
from jax.experimental.pallas import tpu_sc as plsc

_SIZE = 16384
_N = 4096
_SMOOTH = 0.1
_CONF = 1.0 - _SMOOTH
_S = _SMOOTH / (_SIZE - 2)
_C0 = (_SIZE - 2) * _S * _math.log(_S) + _CONF * _math.log(_CONF)

_R = 256
_LANES = 128
_XROWS = _N * (_SIZE // _LANES)

_info = plsc.get_sparse_core_info()
_NC, _NS, _L = _info.num_cores, _info.num_subcores, _info.num_lanes
_NW = _NC * _NS
_BPW = _N // _NW


def _sc_gather_body(x1d_hbm, tgt_hbm, a_hbm, m_hbm,
                    tgt_v, idx_v, g_v, a_v, m_v, sem):
    wid = lax.axis_index("s") * _NC + lax.axis_index("c")
    base = wid * _BPW
    pltpu.sync_copy(tgt_hbm.at[pl.ds(base, _BPW)], tgt_v)
    for c in range(_BPW // _L):
        t = tgt_v[pl.ds(c * _L, _L)]
        ivec = lax.iota(jnp.int32, _L) + (base + c * _L)
        idx_v[pl.ds(c * _L, _L)] = ivec * _SIZE + t
    pltpu.async_copy(x1d_hbm.at[idx_v], g_v, sem).wait()
    for c in range(_BPW // _L):
        g = g_v[pl.ds(c * _L, _L)]
        t = tgt_v[pl.ds(c * _L, _L)]
        m = jnp.where(t != 0, jnp.float32(1.0), jnp.float32(0.0))
        a_v[pl.ds(c * _L, _L)] = m * (
            jnp.float32(_C0) + jnp.float32(_S - _CONF) * g)
        m_v[pl.ds(c * _L, _L)] = m
    pltpu.sync_copy(a_v, a_hbm.at[pl.ds(base, _BPW)])
    pltpu.sync_copy(m_v, m_hbm.at[pl.ds(base, _BPW)])


_sc_gather = pl.kernel(
    _sc_gather_body,
    out_type=(
        jax.ShapeDtypeStruct((_N,), jnp.float32),
        jax.ShapeDtypeStruct((_N,), jnp.float32),
    ),
    mesh=plsc.VectorSubcoreMesh(core_axis_name="c", subcore_axis_name="s"),
    scratch_types=[
        pltpu.VMEM((_BPW,), jnp.int32),
        pltpu.VMEM((_BPW,), jnp.int32),
        pltpu.VMEM((_BPW,), jnp.float32),
        pltpu.VMEM((_BPW,), jnp.float32),
        pltpu.VMEM((_BPW,), jnp.float32),
        pltpu.SemaphoreType.DMA,
    ],
)


def _rowsum_body(x_ref, out_ref):
    x = x_ref[...]
    out_ref[0, 0, :] = jnp.sum(x, axis=1) - x[:, 0]


def _combine_body(a_ref, m_ref, rs_ref, out_ref):
    out_ref[0, 0] = jnp.sum(
        a_ref[...] - jnp.float32(_S) * m_ref[...] * rs_ref[...])


@jax.jit
def kernel(x, target):
    nr = _N // _R
    x1d = x.reshape(_N * _SIZE)
    a, m = _sc_gather(x1d, target.astype(jnp.int32))
    rs = pl.pallas_call(
        _rowsum_body,
        grid=(nr,),
        in_specs=[pl.BlockSpec((_R, _SIZE), lambda i: (i, 0))],
        out_specs=pl.BlockSpec((1, 1, _R), lambda i: (i, 0, 0)),
        out_shape=jax.ShapeDtypeStruct((nr, 1, _R), jnp.float32),
    )(x)
    out = pl.pallas_call(
        _combine_body,
        in_specs=[
            pl.BlockSpec((_NW, _BPW), lambda: (0, 0)),
            pl.BlockSpec((_NW, _BPW), lambda: (0, 0)),
            pl.BlockSpec((_NW, _BPW), lambda: (0, 0)),
        ],
        out_specs=pl.BlockSpec((1, 1), lambda: (0, 0), memory_space=pltpu.SMEM),
        out_shape=jax.ShapeDtypeStruct((1, 1), jnp.float32),
    )(a.reshape(_NW, _BPW), m.reshape(_NW, _BPW), rs.reshape(_NW, _BPW))
    return out[0, 0]


# speedup vs baseline: 2.7929x; 2.7929x over previous
import math as _math

import jax
import jax.numpy as jnp
from jax import lax
from jax.experimental import pallas as pl
from jax.experimental.pallas import tpu as pltpu
from jax.experimental.pallas import tpu_sc as plsc

_SIZE = 16384
_N = 4096
_SMOOTH = 0.1
_CONF = 1.0 - _SMOOTH
_S = _SMOOTH / (_SIZE - 2)
_C0 = (_SIZE - 2) * _S * _math.log(_S) + _CONF * _math.log(_CONF)

_R = 256

_info = plsc.get_sparse_core_info()
_NC, _NS, _L = _info.num_cores, _info.num_subcores, _info.num_lanes
_NW = _NC * _NS
_BPW = _N // _NW


def _tc_body(tgt_ref, x_ref, rs_ref, g_ref):
    t = tgt_ref[0, 0, :]
    x = x_ref[...]
    cols = lax.broadcasted_iota(jnp.int32, (_R, _SIZE), 1)
    onehot = cols == t[:, None]
    g_ref[0, 0, :] = jnp.sum(jnp.where(onehot, x, jnp.float32(0.0)), axis=1)
    rs_ref[0, 0, :] = jnp.sum(x, axis=1) - x[:, 0]


def _sc_combine_body(tgt_hbm, rs_hbm, g_hbm, out_hbm,
                     tgt_v, rs_v, g_v, part_v):
    wid = lax.axis_index("s") * _NC + lax.axis_index("c")
    base = wid * _BPW
    pltpu.sync_copy(tgt_hbm.at[pl.ds(base, _BPW)], tgt_v)
    pltpu.sync_copy(rs_hbm.at[pl.ds(base, _BPW)], rs_v)
    pltpu.sync_copy(g_hbm.at[pl.ds(base, _BPW)], g_v)
    acc = jnp.zeros((_L,), jnp.float32)
    for c in range(_BPW // _L):
        t = tgt_v[pl.ds(c * _L, _L)]
        rs = rs_v[pl.ds(c * _L, _L)]
        g = g_v[pl.ds(c * _L, _L)]
        row = (jnp.float32(_C0) + jnp.float32(_S - _CONF) * g
               - jnp.float32(_S) * rs)
        acc = acc + jnp.where(t != 0, row, jnp.float32(0.0))
    part_v[...] = acc
    pltpu.sync_copy(part_v, out_hbm.at[wid])


_sc_combine = pl.kernel(
    _sc_combine_body,
    out_type=jax.ShapeDtypeStruct((_NW, _L), jnp.float32),
    mesh=plsc.VectorSubcoreMesh(core_axis_name="c", subcore_axis_name="s"),
    scratch_types=[
        pltpu.VMEM((_BPW,), jnp.int32),
        pltpu.VMEM((_BPW,), jnp.float32),
        pltpu.VMEM((_BPW,), jnp.float32),
        pltpu.VMEM((_L,), jnp.float32),
    ],
)


@jax.jit
def kernel(x, target):
    nr = _N // _R
    tgt3 = target.astype(jnp.int32).reshape(nr, 1, _R)
    rs, g = pl.pallas_call(
        _tc_body,
        grid=(nr,),
        in_specs=[
            pl.BlockSpec((1, 1, _R), lambda i: (i, 0, 0)),
            pl.BlockSpec((_R, _SIZE), lambda i: (i, 0)),
        ],
        out_specs=[
            pl.BlockSpec((1, 1, _R), lambda i: (i, 0, 0)),
            pl.BlockSpec((1, 1, _R), lambda i: (i, 0, 0)),
        ],
        out_shape=[
            jax.ShapeDtypeStruct((nr, 1, _R), jnp.float32),
            jax.ShapeDtypeStruct((nr, 1, _R), jnp.float32),
        ],
    )(tgt3, x)
    parts = _sc_combine(target.astype(jnp.int32),
                        rs.reshape(_N), g.reshape(_N))
    return jnp.sum(parts)
